# Initial kernel scaffold; baseline (speedup 1.0000x reference)
import jax
import jax.numpy as jnp
from jax.experimental import pallas as pl


def kernel(x, edge_index, ln_gamma, ln_beta, gine_eps, W, b):
    raise NotImplementedError("write your pallas kernel here")


# speedup vs baseline: 3.3969x; 3.3969x over previous
import functools

import jax
import jax.numpy as jnp
from jax import lax
from jax.experimental import pallas as pl
from jax.experimental.pallas import tpu as pltpu
from jax.experimental.pallas import tpu_sc as plsc

N = 10000
D = 128
E = 320000
NC = 2
NS = 16
NW = NC * NS
CHUNK = 128
CHUNKS = 80
EPT = CHUNK * CHUNKS
EPAD = EPT * NW
NPAD = 10240
ZROWS = NPAD // NS
OROWS = N // NS
LN_EPS = 1e-5


def _ln_relu(x, gamma, beta):
    def body(x_ref, g_ref, b_ref, o_ref):
        xv = x_ref[...]
        mu = jnp.mean(xv, axis=-1, keepdims=True)
        xc = xv - mu
        var = jnp.mean(xc * xc, axis=-1, keepdims=True)
        hh = xc * lax.rsqrt(var + LN_EPS) * g_ref[...] + b_ref[...]
        o_ref[...] = jnp.maximum(hh, 0.0)

    blk = 2000
    return pl.pallas_call(
        body,
        grid=(N // blk,),
        in_specs=[
            pl.BlockSpec((blk, D), lambda i: (i, 0)),
            pl.BlockSpec((1, D), lambda i: (0, 0)),
            pl.BlockSpec((1, D), lambda i: (0, 0)),
        ],
        out_specs=pl.BlockSpec((blk, D), lambda i: (i, 0)),
        out_shape=jax.ShapeDtypeStruct((N, D), jnp.float32),
    )(x, gamma[None, :], beta[None, :])


def _sc_segment_sum(h, srcp, dstp):
    mesh = plsc.VectorSubcoreMesh(core_axis_name="c", subcore_axis_name="s")

    @functools.partial(
        pl.kernel,
        out_type=jax.ShapeDtypeStruct((NC, NPAD, D), jnp.float32),
        mesh=mesh,
        scratch_types=[
            pltpu.VMEM((CHUNKS, CHUNK), jnp.int32),
            pltpu.VMEM((CHUNKS, CHUNK), jnp.int32),
            pltpu.VMEM((CHUNK, D), jnp.float32),
            pltpu.VMEM_SHARED((NPAD, D), jnp.float32),
        ],
    )
    def k(h_hbm, src_hbm, dst_hbm, out_hbm, src_v, dst_v, buf_a, agg):
        c = lax.axis_index("c")
        s = lax.axis_index("s")
        wid = c * NS + s

        pltpu.sync_copy(src_hbm.at[wid], src_v)
        pltpu.sync_copy(dst_hbm.at[wid], dst_v)

        zero = jnp.zeros((16,), jnp.float32)

        def _zrow(r, carry):
            for kk in range(D // 16):
                buf_a[r, pl.ds(kk * 16, 16)] = zero
            return carry

        lax.fori_loop(0, CHUNK, _zrow, 0)
        for j in range(ZROWS // CHUNK):
            pltpu.sync_copy(buf_a, agg.at[pl.ds(s * ZROWS + j * CHUNK, CHUNK)])
        plsc.subcore_barrier()

        def _edge_chunk(i, carry):
            pltpu.sync_copy(h_hbm.at[src_v.at[i]], buf_a)
            pltpu.sync_copy(buf_a, agg.at[dst_v.at[i]], add=True)
            return carry

        lax.fori_loop(0, CHUNKS, _edge_chunk, 0)
        plsc.subcore_barrier()

        pltpu.sync_copy(agg.at[pl.ds(s * ZROWS, ZROWS)],
                        out_hbm.at[c, pl.ds(s * ZROWS, ZROWS)])

    return k(h, srcp, dstp)


def _final(x, parts, gamma, beta, W, b, eps):
    def body(e_ref, x_ref, p_ref, g_ref, be_ref, w_ref, b_ref, o_ref):
        xv = x_ref[...]
        mu = jnp.mean(xv, axis=-1, keepdims=True)
        xc = xv - mu
        var = jnp.mean(xc * xc, axis=-1, keepdims=True)
        hh = jnp.maximum(xc * lax.rsqrt(var + LN_EPS) * g_ref[...] + be_ref[...], 0.0)
        z = (1.0 + e_ref[0]) * hh + p_ref[0] + p_ref[1]
        o = lax.dot_general(z, w_ref[...], (((1,), (1,)), ((), ())),
                            preferred_element_type=jnp.float32)
        o_ref[...] = o + b_ref[...] + xv

    blk = 2000
    return pl.pallas_call(
        body,
        grid=(N // blk,),
        in_specs=[
            pl.BlockSpec(memory_space=pltpu.SMEM),
            pl.BlockSpec((blk, D), lambda i: (i, 0)),
            pl.BlockSpec((NC, blk, D), lambda i: (0, i, 0)),
            pl.BlockSpec((1, D), lambda i: (0, 0)),
            pl.BlockSpec((1, D), lambda i: (0, 0)),
            pl.BlockSpec((D, D), lambda i: (0, 0)),
            pl.BlockSpec((1, D), lambda i: (0, 0)),
        ],
        out_specs=pl.BlockSpec((blk, D), lambda i: (i, 0)),
        out_shape=jax.ShapeDtypeStruct((N, D), jnp.float32),
    )(eps.reshape(1), x, parts, gamma[None, :], beta[None, :], W, b[None, :])


def kernel(x, edge_index, ln_gamma, ln_beta, gine_eps, W, b):
    h = _ln_relu(x, ln_gamma, ln_beta)
    pad = EPAD - E
    srcp = jnp.concatenate([edge_index[0], jnp.zeros((pad,), jnp.int32)])
    dstp = jnp.concatenate([edge_index[1], jnp.full((pad,), N, jnp.int32)])
    srcp = srcp.reshape(NW, CHUNKS, CHUNK)
    dstp = dstp.reshape(NW, CHUNKS, CHUNK)
    parts = _sc_segment_sum(h, srcp, dstp)[:, :N, :]
    return _final(x, parts, ln_gamma, ln_beta, W, b, gine_eps)


# speedup vs baseline: 3.7859x; 1.1145x over previous
import functools

import jax
import jax.numpy as jnp
from jax import lax
from jax.experimental import pallas as pl
from jax.experimental.pallas import tpu as pltpu
from jax.experimental.pallas import tpu_sc as plsc

N = 10000
D = 128
E = 320000
NC = 2
NS = 16
NW = NC * NS
CHUNK = 128
CHUNKS = 80
HALF = CHUNKS // 2
EPT = CHUNK * CHUNKS
EPAD = EPT * NW
NPAD = 10240
ZROWS = NPAD // NS
OROWS = N // NS
LN_EPS = 1e-5


def _ln_relu(x, gamma, beta):
    def body(x_ref, g_ref, b_ref, o_ref):
        xv = x_ref[...]
        mu = jnp.mean(xv, axis=-1, keepdims=True)
        xc = xv - mu
        var = jnp.mean(xc * xc, axis=-1, keepdims=True)
        hh = xc * lax.rsqrt(var + LN_EPS) * g_ref[...] + b_ref[...]
        o_ref[...] = jnp.maximum(hh, 0.0)

    blk = 2000
    return pl.pallas_call(
        body,
        grid=(N // blk,),
        in_specs=[
            pl.BlockSpec((blk, D), lambda i: (i, 0)),
            pl.BlockSpec((1, D), lambda i: (0, 0)),
            pl.BlockSpec((1, D), lambda i: (0, 0)),
        ],
        out_specs=pl.BlockSpec((blk, D), lambda i: (i, 0)),
        out_shape=jax.ShapeDtypeStruct((N, D), jnp.float32),
    )(x, gamma[None, :], beta[None, :])


def _sc_segment_sum(h, srcp, dstp):
    mesh = plsc.VectorSubcoreMesh(core_axis_name="c", subcore_axis_name="s")

    @functools.partial(
        pl.kernel,
        out_type=jax.ShapeDtypeStruct((NC, NPAD, D), jnp.float32),
        mesh=mesh,
        scratch_types=[
            pltpu.VMEM((HALF, CHUNK), jnp.int32),
            pltpu.VMEM((HALF, CHUNK), jnp.int32),
            pltpu.VMEM((CHUNK, D), jnp.float32),
            pltpu.VMEM((CHUNK, D), jnp.float32),
            pltpu.VMEM_SHARED((NPAD, D), jnp.float32),
            pltpu.SemaphoreType.DMA,
            pltpu.SemaphoreType.DMA,
        ],
    )
    def k(h_hbm, src_hbm, dst_hbm, out_hbm, src_v, dst_v, buf_a, buf_b, agg,
          sem_a, sem_b):
        c = lax.axis_index("c")
        s = lax.axis_index("s")
        wid = c * NS + s

        zero = jnp.zeros((16,), jnp.float32)

        def _zrow(r, carry):
            for kk in range(D // 16):
                buf_a[r, pl.ds(kk * 16, 16)] = zero
            return carry

        lax.fori_loop(0, CHUNK, _zrow, 0)
        for j in range(ZROWS // CHUNK):
            pltpu.sync_copy(buf_a, agg.at[pl.ds(s * ZROWS + j * CHUNK, CHUNK)])
        plsc.subcore_barrier()

        def _step(i, buf, sem):
            pltpu.make_async_copy(h_hbm.at[src_v.at[i]], buf, sem).wait()
            pltpu.sync_copy(buf, agg.at[dst_v.at[i]], add=True)

            @pl.when(i + 2 < HALF)
            def _():
                pltpu.async_copy(h_hbm.at[src_v.at[i + 2]], buf, sem)

        for hb in range(2):
            pltpu.sync_copy(src_hbm.at[wid, pl.ds(hb * HALF, HALF)], src_v)
            pltpu.sync_copy(dst_hbm.at[wid, pl.ds(hb * HALF, HALF)], dst_v)
            pltpu.async_copy(h_hbm.at[src_v.at[0]], buf_a, sem_a)
            pltpu.async_copy(h_hbm.at[src_v.at[1]], buf_b, sem_b)

            def _pair(g, carry):
                _step(2 * g, buf_a, sem_a)
                _step(2 * g + 1, buf_b, sem_b)
                return carry

            lax.fori_loop(0, HALF // 2, _pair, 0)
        plsc.subcore_barrier()

        pltpu.sync_copy(agg.at[pl.ds(s * ZROWS, ZROWS)],
                        out_hbm.at[c, pl.ds(s * ZROWS, ZROWS)])

    return k(h, srcp, dstp)


def _final(x, parts, gamma, beta, W, b, eps):
    def body(e_ref, x_ref, p_ref, g_ref, be_ref, w_ref, b_ref, o_ref):
        xv = x_ref[...]
        mu = jnp.mean(xv, axis=-1, keepdims=True)
        xc = xv - mu
        var = jnp.mean(xc * xc, axis=-1, keepdims=True)
        hh = jnp.maximum(xc * lax.rsqrt(var + LN_EPS) * g_ref[...] + be_ref[...], 0.0)
        z = (1.0 + e_ref[0]) * hh + p_ref[0] + p_ref[1]
        o = lax.dot_general(z, w_ref[...], (((1,), (1,)), ((), ())),
                            preferred_element_type=jnp.float32)
        o_ref[...] = o + b_ref[...] + xv

    blk = 2000
    return pl.pallas_call(
        body,
        grid=(N // blk,),
        in_specs=[
            pl.BlockSpec(memory_space=pltpu.SMEM),
            pl.BlockSpec((blk, D), lambda i: (i, 0)),
            pl.BlockSpec((NC, blk, D), lambda i: (0, i, 0)),
            pl.BlockSpec((1, D), lambda i: (0, 0)),
            pl.BlockSpec((1, D), lambda i: (0, 0)),
            pl.BlockSpec((D, D), lambda i: (0, 0)),
            pl.BlockSpec((1, D), lambda i: (0, 0)),
        ],
        out_specs=pl.BlockSpec((blk, D), lambda i: (i, 0)),
        out_shape=jax.ShapeDtypeStruct((N, D), jnp.float32),
    )(eps.reshape(1), x, parts, gamma[None, :], beta[None, :], W, b[None, :])


def kernel(x, edge_index, ln_gamma, ln_beta, gine_eps, W, b):
    h = _ln_relu(x, ln_gamma, ln_beta)
    pad = EPAD - E
    srcp = jnp.concatenate([edge_index[0], jnp.zeros((pad,), jnp.int32)])
    dstp = jnp.concatenate([edge_index[1], jnp.full((pad,), N, jnp.int32)])
    srcp = srcp.reshape(NW, CHUNKS, CHUNK)
    dstp = dstp.reshape(NW, CHUNKS, CHUNK)
    parts = _sc_segment_sum(h, srcp, dstp)[:, :N, :]
    return _final(x, parts, ln_gamma, ln_beta, W, b, gine_eps)
